# Initial kernel scaffold; baseline (speedup 1.0000x reference)
import jax
import jax.numpy as jnp
from jax.experimental import pallas as pl


def kernel(x, edge_index, batch, W1, b1, W2, b2, lin1_w, lin1_b, lin2_w, lin2_b):
    raise NotImplementedError("write your pallas kernel here")


# speedup vs baseline: 5.2814x; 5.2814x over previous
import functools

import jax
import jax.numpy as jnp
from jax import lax
from jax.experimental import pallas as pl
from jax.experimental.pallas import tpu as pltpu
from jax.experimental.pallas import tpu_sc as plsc

N = 10000
NP = 10240
E = 320000
D = 128
G = 16
NPG = N // G

NC = 2
NS = 16
NW = NC * NS
EPW = E // NW
K = 80
NCHUNK = EPW // K
RPT = NP // NS


def _sc_agg_body(h_hbm, src_hbm, dst_hbm, zrows_hbm, out_hbm,
                 agg_sh, src_v, dst_v, rows_v, sem):
    cid = lax.axis_index("c")
    sid = lax.axis_index("s")
    wid = sid * NC + cid
    base = wid * EPW

    pltpu.sync_copy(zrows_hbm, agg_sh.at[pl.ds(sid * RPT, RPT)])
    plsc.subcore_barrier()

    def body(i, _):
        off = base + i * K
        pltpu.sync_copy(src_hbm.at[pl.ds(off, K)], src_v)
        pltpu.sync_copy(dst_hbm.at[pl.ds(off, K)], dst_v)
        pltpu.async_copy(h_hbm.at[src_v], rows_v, sem).wait()
        pltpu.sync_copy(rows_v, agg_sh.at[dst_v], add=True)
        return 0

    lax.fori_loop(0, NCHUNK, body, 0)
    plsc.subcore_barrier()
    pltpu.sync_copy(agg_sh.at[pl.ds(sid * RPT, RPT)],
                    out_hbm.at[cid, pl.ds(sid * RPT, RPT)])


_sc_agg = pl.kernel(
    _sc_agg_body,
    out_type=jax.ShapeDtypeStruct((NC, NP, D), jnp.float32),
    mesh=plsc.VectorSubcoreMesh(core_axis_name="c", subcore_axis_name="s"),
    scratch_types=[
        pltpu.VMEM_SHARED((NP, D), jnp.float32),
        pltpu.VMEM((K,), jnp.int32),
        pltpu.VMEM((K,), jnp.int32),
        pltpu.VMEM((K, D), jnp.float32),
        pltpu.SemaphoreType.DMA,
    ],
)


def _tc_layer_body(p_ref, w_ref, b_ref, o_ref):
    a = p_ref[0] + p_ref[1]
    z = jnp.dot(a, w_ref[...], preferred_element_type=jnp.float32) + b_ref[...]
    o_ref[...] = jnp.maximum(z, 0.2 * z)


def _tc_layer(partials, w, b):
    R = 2048
    return pl.pallas_call(
        _tc_layer_body,
        out_shape=jax.ShapeDtypeStruct((NP, D), jnp.float32),
        grid=(NP // R,),
        in_specs=[
            pl.BlockSpec((NC, R, D), lambda i: (0, i, 0)),
            pl.BlockSpec((D, D), lambda i: (0, 0)),
            pl.BlockSpec((1, D), lambda i: (0, 0)),
        ],
        out_specs=pl.BlockSpec((R, D), lambda i: (i, 0)),
    )(partials, w, b.reshape(1, D))


def _tc_pool_body(p_ref, w_ref, b_ref, o_ref):
    a = p_ref[0] + p_ref[1]
    z = jnp.dot(a, w_ref[...], preferred_element_type=jnp.float32) + b_ref[...]
    h = jnp.maximum(z, 0.2 * z)
    hh = h.reshape(-1, NPG, D)
    o_ref[...] = jnp.sum(hh, axis=1) * (1.0 / NPG)


def _tc_pool(partials, w, b):
    GB = 8
    R = GB * NPG
    return pl.pallas_call(
        _tc_pool_body,
        out_shape=jax.ShapeDtypeStruct((G, D), jnp.float32),
        grid=(G // GB,),
        in_specs=[
            pl.BlockSpec((NC, R, D), lambda i: (0, i, 0)),
            pl.BlockSpec((D, D), lambda i: (0, 0)),
            pl.BlockSpec((1, D), lambda i: (0, 0)),
        ],
        out_specs=pl.BlockSpec((GB, D), lambda i: (i, 0)),
    )(partials, w, b.reshape(1, D))


def _tc_mlp_body(p_ref, w1_ref, b1_ref, w2_ref, b2_ref, o_ref):
    z = jnp.dot(p_ref[...], w1_ref[...], preferred_element_type=jnp.float32)
    z = z + b1_ref[...]
    g = jnp.maximum(z, 0.2 * z)
    o_ref[...] = jnp.dot(g, w2_ref[...],
                         preferred_element_type=jnp.float32) + b2_ref[...]


def _tc_mlp(pooled, w1, b1, w2, b2):
    C = w2.shape[1]
    H2 = w1.shape[1]
    return pl.pallas_call(
        _tc_mlp_body,
        out_shape=jax.ShapeDtypeStruct((G, C), jnp.float32),
    )(pooled, w1, b1.reshape(1, H2), w2, b2.reshape(1, C))


def kernel(x, edge_index, batch, W1, b1, W2, b2, lin1_w, lin1_b, lin2_w, lin2_b):
    src = edge_index[0]
    dst = edge_index[1]
    zrows = jnp.zeros((RPT, D), jnp.float32)

    p1 = _sc_agg(x, src, dst, zrows)
    h1 = _tc_layer(p1, W1, b1)
    p2 = _sc_agg(h1, src, dst, zrows)
    pooled = _tc_pool(p2, W2 + W2, b2)
    return _tc_mlp(pooled, lin1_w, lin1_b, lin2_w, lin2_b)
